# baseline (speedup 1.0000x reference)
import jax
import jax.numpy as jnp
from jax.experimental import pallas as pl

NUM_EXPERTS = 64
TOP_K = 8
BLOCK_ROWS = 512


def _router_kernel(hs_ref, wt_ref, wtt_ref, logits_ref, idx_ref, w_ref):
    hs = hs_ref[:]
    logits_ref[:] = jnp.dot(hs, wt_ref[:], preferred_element_type=jnp.float32)

    work = jax.lax.dot_general(
        wtt_ref[:], hs, (((1,), (1,)), ((), ())),
        preferred_element_type=jnp.float32)

    rows = work.shape[1]
    iota = jax.lax.broadcasted_iota(jnp.int32, (NUM_EXPERTS, rows), 0)
    vals = []
    idxs = []
    for _ in range(TOP_K):
        m = jnp.max(work, axis=0, keepdims=True)
        cand = jnp.where(work == m, iota, NUM_EXPERTS)
        idx = jnp.min(cand, axis=0, keepdims=True)
        vals.append(m)
        idxs.append(idx)
        work = jnp.where(cand == idx, -jnp.inf, work)

    topv = jnp.concatenate(vals, axis=0)
    topi = jnp.concatenate(idxs, axis=0)
    e = jnp.exp(topv - topv[0:1, :])
    w_ref[:] = (e / jnp.sum(e, axis=0, keepdims=True)).T
    idx_ref[:] = topi.T


@jax.jit
def _router(hs, wt, wtt):
    n = hs.shape[0]
    grid = (n // BLOCK_ROWS,)
    return pl.pallas_call(
        _router_kernel,
        grid=grid,
        in_specs=[
            pl.BlockSpec((BLOCK_ROWS, hs.shape[1]), lambda i: (i, 0)),
            pl.BlockSpec((hs.shape[1], NUM_EXPERTS), lambda i: (0, 0)),
            pl.BlockSpec((NUM_EXPERTS, hs.shape[1]), lambda i: (0, 0)),
        ],
        out_specs=[
            pl.BlockSpec((BLOCK_ROWS, NUM_EXPERTS), lambda i: (i, 0)),
            pl.BlockSpec((BLOCK_ROWS, TOP_K), lambda i: (i, 0)),
            pl.BlockSpec((BLOCK_ROWS, TOP_K), lambda i: (i, 0)),
        ],
        out_shape=[
            jax.ShapeDtypeStruct((n, NUM_EXPERTS), jnp.float32),
            jax.ShapeDtypeStruct((n, TOP_K), jnp.int32),
            jax.ShapeDtypeStruct((n, TOP_K), jnp.float32),
        ],
    )(hs, wt, wtt)


def kernel(hidden_states, weight):
    hs = hidden_states.reshape(-1, hidden_states.shape[-1]).astype(jnp.float32)
    wt = weight.astype(jnp.float32)
    logits, topi, topw = _router(hs, wt.T, wt)
    return (topi, topw, logits)


# speedup vs baseline: 1.2742x; 1.2742x over previous
import jax
import jax.numpy as jnp
from jax.experimental import pallas as pl

NUM_EXPERTS = 64
TOP_K = 8
BLOCK_ROWS = 1024


def _router_kernel(hs_ref, wt_ref, wtt_ref, logits_ref, idx_ref, w_ref):
    hs = hs_ref[:]
    logits_ref[:] = jnp.dot(hs, wt_ref[:], preferred_element_type=jnp.float32)

    work = jax.lax.dot_general(
        wtt_ref[:], hs, (((1,), (1,)), ((), ())),
        preferred_element_type=jnp.float32)

    rows = work.shape[1]
    iota = jax.lax.broadcasted_iota(jnp.int32, (NUM_EXPERTS, rows), 0)
    vals = []
    idxs = []
    for _ in range(TOP_K):
        m = jnp.max(work, axis=0, keepdims=True)
        cand = jnp.where(work == m, iota, NUM_EXPERTS)
        idx = jnp.min(cand, axis=0, keepdims=True)
        vals.append(m)
        idxs.append(idx)
        work = jnp.where(cand == idx, -jnp.inf, work)

    topv = jnp.concatenate(vals, axis=0)
    topi = jnp.concatenate(idxs, axis=0)
    e = jnp.exp(topv - topv[0:1, :])
    w_ref[:] = e / jnp.sum(e, axis=0, keepdims=True)
    idx_ref[:] = topi


@jax.jit
def _router(hs, wt, wtt):
    n = hs.shape[0]
    grid = (n // BLOCK_ROWS,)
    return pl.pallas_call(
        _router_kernel,
        grid=grid,
        in_specs=[
            pl.BlockSpec((BLOCK_ROWS, hs.shape[1]), lambda i: (i, 0)),
            pl.BlockSpec((hs.shape[1], NUM_EXPERTS), lambda i: (0, 0)),
            pl.BlockSpec((NUM_EXPERTS, hs.shape[1]), lambda i: (0, 0)),
        ],
        out_specs=[
            pl.BlockSpec((BLOCK_ROWS, NUM_EXPERTS), lambda i: (i, 0)),
            pl.BlockSpec((TOP_K, BLOCK_ROWS), lambda i: (0, i)),
            pl.BlockSpec((TOP_K, BLOCK_ROWS), lambda i: (0, i)),
        ],
        out_shape=[
            jax.ShapeDtypeStruct((n, NUM_EXPERTS), jnp.float32),
            jax.ShapeDtypeStruct((TOP_K, n), jnp.int32),
            jax.ShapeDtypeStruct((TOP_K, n), jnp.float32),
        ],
    )(hs, wt, wtt)


def kernel(hidden_states, weight):
    hs = hidden_states.reshape(-1, hidden_states.shape[-1]).astype(jnp.float32)
    wt = weight.astype(jnp.float32)
    logits, topi, topw = _router(hs, wt.T, wt)
    return (topi.T, topw.T, logits)


# speedup vs baseline: 1.3980x; 1.0972x over previous
import jax
import jax.numpy as jnp
from jax.experimental import pallas as pl

NUM_EXPERTS = 64
TOP_K = 8
BLOCK_ROWS = 1024


def _router_kernel(hs_ref, wt_ref, wtt_ref, logits_ref, idx_ref, w_ref):
    hs = hs_ref[:]
    logits = jnp.dot(hs, wt_ref[:], preferred_element_type=jnp.float32)
    logits_ref[:] = logits

    work = logits.T

    rows = work.shape[1]
    iota = jax.lax.broadcasted_iota(jnp.int32, (NUM_EXPERTS, rows), 0)
    vals = []
    idxs = []
    for _ in range(TOP_K):
        m = jnp.max(work, axis=0, keepdims=True)
        cand = jnp.where(work == m, iota, NUM_EXPERTS)
        idx = jnp.min(cand, axis=0, keepdims=True)
        vals.append(m)
        idxs.append(idx)
        work = jnp.where(cand == idx, -jnp.inf, work)

    topv = jnp.concatenate(vals, axis=0)
    topi = jnp.concatenate(idxs, axis=0)
    e = jnp.exp(topv - topv[0:1, :])
    w_ref[:] = e / jnp.sum(e, axis=0, keepdims=True)
    idx_ref[:] = topi


@jax.jit
def _router(hs, wt, wtt):
    n = hs.shape[0]
    grid = (n // BLOCK_ROWS,)
    return pl.pallas_call(
        _router_kernel,
        grid=grid,
        in_specs=[
            pl.BlockSpec((BLOCK_ROWS, hs.shape[1]), lambda i: (i, 0)),
            pl.BlockSpec((hs.shape[1], NUM_EXPERTS), lambda i: (0, 0)),
            pl.BlockSpec((NUM_EXPERTS, hs.shape[1]), lambda i: (0, 0)),
        ],
        out_specs=[
            pl.BlockSpec((BLOCK_ROWS, NUM_EXPERTS), lambda i: (i, 0)),
            pl.BlockSpec((TOP_K, BLOCK_ROWS), lambda i: (0, i)),
            pl.BlockSpec((TOP_K, BLOCK_ROWS), lambda i: (0, i)),
        ],
        out_shape=[
            jax.ShapeDtypeStruct((n, NUM_EXPERTS), jnp.float32),
            jax.ShapeDtypeStruct((TOP_K, n), jnp.int32),
            jax.ShapeDtypeStruct((TOP_K, n), jnp.float32),
        ],
    )(hs, wt, wtt)


def kernel(hidden_states, weight):
    hs = hidden_states.reshape(-1, hidden_states.shape[-1]).astype(jnp.float32)
    wt = weight.astype(jnp.float32)
    logits, topi, topw = _router(hs, wt.T, wt)
    return (topi.T, topw.T, logits)
